# baseline (speedup 1.0000x reference)
import functools

import jax
import jax.numpy as jnp
from jax import lax
from jax.experimental import pallas as pl
from jax.experimental.pallas import tpu as pltpu
from jax.experimental.pallas import tpu_sc as plsc

_B = 16384
_D = 32
_NC = 2
_NS = 16
_NW = _NC * _NS
_BPW = _B // _NW
_IC = 128
_NCHUNK = _BPW // _IC
_L = 16
_NSEM = 4


def _sc_body(heads_hbm, tails_hbm, rels_hbm, ent_hbm, rel_hbm, out_hbm,
             hidx, tidx, ridx, e1c, e2c, rc, out_v, *sems):
    wid = lax.axis_index("s") * _NC + lax.axis_index("c")
    row0 = wid * _NCHUNK
    pltpu.sync_copy(heads_hbm.at[pl.ds(row0, _NCHUNK)], hidx)
    pltpu.sync_copy(tails_hbm.at[pl.ds(row0, _NCHUNK)], tidx)
    pltpu.sync_copy(rels_hbm.at[pl.ds(row0, _NCHUNK)], ridx)

    pending = {}

    def fire(d):
        sem = sems[d % _NSEM]
        cs = []
        for j in range(_NCHUNK):
            dst = pl.ds(j * _IC, _IC)
            cs.append(pltpu.async_copy(ent_hbm.at[d].at[hidx.at[j]],
                                       e1c.at[d, dst], sem))
            cs.append(pltpu.async_copy(ent_hbm.at[d].at[tidx.at[j]],
                                       e2c.at[d, dst], sem))
            cs.append(pltpu.async_copy(rel_hbm.at[d].at[ridx.at[j]],
                                       rc.at[d, dst], sem))
        pending[d] = cs

    for d in range(_D):
        fire(d)
        if d >= _NSEM - 1:
            for c in pending.pop(d - (_NSEM - 1)):
                c.wait()
    for d in range(_D - (_NSEM - 1), _D):
        for c in pending.pop(d):
            c.wait()

    def group(g, carry):
        sl = pl.ds(pl.multiple_of(g * _L, _L), _L)
        acc = jnp.zeros((_L,), jnp.float32)
        for d in range(_D):
            acc = acc + e1c[d, sl] * rc[d, sl] * e2c[d, sl]
        pred = 1.0 / (1.0 + jnp.exp(-acc))
        out_v[sl] = pred
        return carry

    lax.fori_loop(0, _BPW // _L, group, 0)
    pltpu.sync_copy(out_v, out_hbm.at[pl.ds(wid * _BPW, _BPW)])


_sc_call = functools.partial(
    pl.kernel,
    out_type=jax.ShapeDtypeStruct((_B,), jnp.float32),
    mesh=plsc.VectorSubcoreMesh(core_axis_name="c", subcore_axis_name="s"),
    compiler_params=pltpu.CompilerParams(
        use_tc_tiling_on_sc=False, needs_layout_passes=False
    ),
    scratch_types=[
        pltpu.VMEM((_NCHUNK, _IC), jnp.int32),
        pltpu.VMEM((_NCHUNK, _IC), jnp.int32),
        pltpu.VMEM((_NCHUNK, _IC), jnp.int32),
        pltpu.VMEM((_D, _BPW), jnp.float32),
        pltpu.VMEM((_D, _BPW), jnp.float32),
        pltpu.VMEM((_D, _BPW), jnp.float32),
        pltpu.VMEM((_BPW,), jnp.float32),
    ] + [pltpu.SemaphoreType.DMA] * _NSEM,
)(_sc_body)


@jax.jit
def kernel(heads, tails, relations, entity_embedding, relation_embedding):
    h2 = heads.reshape(_B // _IC, _IC)
    t2 = tails.reshape(_B // _IC, _IC)
    r2 = relations.reshape(_B // _IC, _IC)
    return _sc_call(h2, t2, r2, entity_embedding.T, relation_embedding.T)


# speedup vs baseline: 18.4181x; 18.4181x over previous
import functools

import jax
import jax.numpy as jnp
from jax import lax
from jax.experimental import pallas as pl
from jax.experimental.pallas import tpu as pltpu
from jax.experimental.pallas import tpu_sc as plsc

_B = 16384
_D = 32
_NC = 2
_NS = 16
_NW = _NC * _NS
_BPW = _B // _NW
_IC = 128
_NCHUNK = _BPW // _IC
_L = 16
_G = _BPW // _L
_NSEM = 4

_NE = 1000000
_NR = 1000
_TCE = (_NE + 127) // 128
_TCR = (_NR + 127) // 128


def _sc_body(heads_hbm, tails_hbm, rels_hbm, ent_hbm, rel_hbm, out_hbm,
             hidx, tidx, ridx, hbase, tbase, rbase, e1c, e2c, relv, out_v,
             rsem, *sems):
    wid = lax.axis_index("s") * _NC + lax.axis_index("c")
    row0 = wid * _NCHUNK
    rel_copy = pltpu.async_copy(rel_hbm, relv, rsem)
    pltpu.sync_copy(heads_hbm.at[pl.ds(row0, _NCHUNK)], hidx)
    pltpu.sync_copy(tails_hbm.at[pl.ds(row0, _NCHUNK)], tidx)
    pltpu.sync_copy(rels_hbm.at[pl.ds(row0, _NCHUNK)], ridx)

    lanemask = jnp.full((_L,), 127, jnp.int32)
    for j in range(_NCHUNK):
        for v in range(_IC // _L):
            sl = pl.ds(v * _L, _L)
            h = hidx[j, sl]
            t = tidx[j, sl]
            r = ridx[j, sl]
            hbase[j, sl] = lax.shift_left(lax.shift_right_logical(h, 7), 10) + (h & lanemask)
            tbase[j, sl] = lax.shift_left(lax.shift_right_logical(t, 7), 10) + (t & lanemask)
            rbase[j, sl] = lax.shift_left(lax.shift_right_logical(r, 7), 10) + (r & lanemask)

    pending = {}

    def fire(d):
        sem = sems[d % _NSEM]
        off_e = (d // 8) * (_TCE * 1024) + (d % 8) * 128
        ent_d = ent_hbm.at[pl.ds(off_e, (_TCE - 1) * 1024 + 128)]
        cs = []
        for j in range(_NCHUNK):
            dst = pl.ds(j * _IC, _IC)
            cs.append(pltpu.async_copy(ent_d.at[hbase.at[j]], e1c.at[d, dst], sem))
            cs.append(pltpu.async_copy(ent_d.at[tbase.at[j]], e2c.at[d, dst], sem))
        pending[d] = cs

    for d in range(_NSEM - 1):
        fire(d)
    rel_copy.wait()

    for d in range(_D):
        if d + _NSEM - 1 < _D:
            fire(d + _NSEM - 1)
        for c in pending.pop(d):
            c.wait()

        off_r = jnp.int32((d // 8) * (_TCR * 1024) + (d % 8) * 128)

        def contrib(g, carry):
            sl = pl.ds(pl.multiple_of(g * _L, _L), _L)
            jj = g // (_IC // _L)
            vv = g % (_IC // _L)
            rb = rbase[jj, pl.ds(pl.multiple_of(vv * _L, _L), _L)] + off_r
            rvals = plsc.load_gather(relv, [rb])
            prod = e1c[d, sl] * rvals * e2c[d, sl]
            if d == 0:
                out_v[sl] = prod
            else:
                out_v[sl] = out_v[sl] + prod
            return carry

        lax.fori_loop(0, _G, contrib, 0)

    def finish(g, carry):
        sl = pl.ds(pl.multiple_of(g * _L, _L), _L)
        out_v[sl] = 1.0 / (1.0 + jnp.exp(-out_v[sl]))
        return carry

    lax.fori_loop(0, _G, finish, 0)
    pltpu.sync_copy(out_v, out_hbm.at[pl.ds(wid * _BPW, _BPW)])


_sc_call = functools.partial(
    pl.kernel,
    out_type=jax.ShapeDtypeStruct((_B,), jnp.float32),
    mesh=plsc.VectorSubcoreMesh(core_axis_name="c", subcore_axis_name="s"),
    compiler_params=pltpu.CompilerParams(
        use_tc_tiling_on_sc=False, needs_layout_passes=False
    ),
    scratch_types=[
        pltpu.VMEM((_NCHUNK, _IC), jnp.int32),
        pltpu.VMEM((_NCHUNK, _IC), jnp.int32),
        pltpu.VMEM((_NCHUNK, _IC), jnp.int32),
        pltpu.VMEM((_NCHUNK, _IC), jnp.int32),
        pltpu.VMEM((_NCHUNK, _IC), jnp.int32),
        pltpu.VMEM((_NCHUNK, _IC), jnp.int32),
        pltpu.VMEM((_D, _BPW), jnp.float32),
        pltpu.VMEM((_D, _BPW), jnp.float32),
        pltpu.VMEM(((_D // 8) * _TCR * 1024,), jnp.float32),
        pltpu.VMEM((_BPW,), jnp.float32),
        pltpu.SemaphoreType.DMA,
    ] + [pltpu.SemaphoreType.DMA] * _NSEM,
)(_sc_body)


def _flat_tiled(table, n_tilecols):
    padded = jnp.pad(table, ((0, n_tilecols * 128 - table.shape[0]), (0, 0)))
    return (padded.T.reshape(_D // 8, 8, n_tilecols, 128)
            .transpose(0, 2, 1, 3).reshape(-1))


@jax.jit
def kernel(heads, tails, relations, entity_embedding, relation_embedding):
    h2 = heads.reshape(_B // _IC, _IC)
    t2 = tails.reshape(_B // _IC, _IC)
    r2 = relations.reshape(_B // _IC, _IC)
    ent = _flat_tiled(entity_embedding, _TCE)
    rel = _flat_tiled(relation_embedding, _TCR)
    return _sc_call(h2, t2, r2, ent, rel)
